# Initial kernel scaffold; baseline (speedup 1.0000x reference)
import jax
import jax.numpy as jnp
from jax.experimental import pallas as pl


def kernel(users, pos, neg, user_emb, item_emb, adj_src, adj_dst, adj_val):
    raise NotImplementedError("write your pallas kernel here")


# speedup vs baseline: 3.2260x; 3.2260x over previous
import functools

import jax
import jax.numpy as jnp
from jax import lax
from jax.experimental import pallas as pl
from jax.experimental.pallas import tpu as pltpu
from jax.experimental.pallas import tpu_sc as plsc

N_USERS = 5000
N_ITEMS = 5000
N_NODES = N_USERS + N_ITEMS
EMB_DIM = 128
N_LAYERS = 3
N_EDGES = 320000
BATCH = 4096

NC = 2
NS = 16
NW = NC * NS
K = 128
CHUNKS_PER_W = -(-N_EDGES // (NW * K))
E_PAD = CHUNKS_PER_W * NW * K
PER_W = CHUNKS_PER_W * K
N_PAD = 10240
TILE_ROWS = N_PAD // NS

_mesh = plsc.VectorSubcoreMesh(core_axis_name="c", subcore_axis_name="s",
                               num_cores=NC, num_subcores=NS)


@functools.partial(
    pl.kernel,
    out_type=jax.ShapeDtypeStruct((NC, N_NODES, EMB_DIM), jnp.float32),
    mesh=_mesh,
    scratch_types=[
        pltpu.VMEM((K,), jnp.int32),
        pltpu.VMEM((K,), jnp.int32),
        pltpu.VMEM((K,), jnp.float32),
        pltpu.VMEM((K, EMB_DIM), jnp.float32),
        pltpu.VMEM_SHARED((N_PAD, EMB_DIM), jnp.float32),
        pltpu.SemaphoreType.DMA,
    ],
)
def _sc_propagate(x_hbm, src_hbm, dst_hbm, val_hbm, out_hbm,
                  src_v, dst_v, val_v, rows_v, partial, sem):
    c = lax.axis_index("c")
    s = lax.axis_index("s")
    wid = c * NS + s

    zeros16 = jnp.zeros((16,), jnp.float32)

    def _zero_row(r, _):
        for j in range(EMB_DIM // 16):
            rows_v[r, pl.ds(j * 16, 16)] = zeros16
        return 0

    lax.fori_loop(0, K, _zero_row, 0)
    r0 = s * TILE_ROWS
    for kk in range(TILE_ROWS // K):
        pltpu.sync_copy(rows_v, partial.at[pl.ds(r0 + kk * K, K)])
    plsc.subcore_barrier()

    base = wid * PER_W

    def _chunk(g, _):
        e0 = base + g * K
        pltpu.sync_copy(src_hbm.at[pl.ds(e0, K)], src_v)
        pltpu.sync_copy(dst_hbm.at[pl.ds(e0, K)], dst_v)
        pltpu.sync_copy(val_hbm.at[pl.ds(e0, K)], val_v)
        pltpu.async_copy(x_hbm.at[src_v], rows_v, sem).wait()

        def _scale(g, _):
            vals = val_v[pl.ds(g * 16, 16)]
            for el in range(16):
                r = g * 16 + el
                v = vals[el]
                for j in range(EMB_DIM // 16):
                    sl = (r, pl.ds(j * 16, 16))
                    rows_v[sl] = rows_v[sl] * v
            return 0

        lax.fori_loop(0, K // 16, _scale, 0)
        pltpu.sync_copy(rows_v, partial.at[dst_v], add=True)
        return 0

    lax.fori_loop(0, CHUNKS_PER_W, _chunk, 0)
    plsc.subcore_barrier()

    for kk in range(TILE_ROWS // K):
        off = r0 + kk * K

        @pl.when(off + K <= N_NODES)
        def _():
            pltpu.sync_copy(partial.at[pl.ds(off, K)],
                            out_hbm.at[c, pl.ds(off, K)])

        rem = N_NODES % K
        if rem:

            @pl.when(off == N_NODES - rem)
            def _():
                pltpu.sync_copy(partial.at[pl.ds(off, rem)],
                                out_hbm.at[c, pl.ds(off, rem)])


GATH_TOTAL = 3 * BATCH
GATH_PER_W = GATH_TOTAL // NW


@functools.partial(
    pl.kernel,
    out_type=jax.ShapeDtypeStruct((GATH_TOTAL, EMB_DIM), jnp.float32),
    mesh=_mesh,
    scratch_types=[
        pltpu.VMEM((K,), jnp.int32),
        pltpu.VMEM((K, EMB_DIM), jnp.float32),
        pltpu.SemaphoreType.DMA,
    ],
)
def _sc_gather(final_hbm, idx_hbm, out_hbm, idx_v, rows_v, sem):
    c = lax.axis_index("c")
    s = lax.axis_index("s")
    base = (c * NS + s) * GATH_PER_W
    for kk in range(GATH_PER_W // K):
        b = base + kk * K
        pltpu.sync_copy(idx_hbm.at[pl.ds(b, K)], idx_v)
        pltpu.async_copy(final_hbm.at[idx_v], rows_v, sem).wait()
        pltpu.sync_copy(rows_v, out_hbm.at[pl.ds(b, K)])


def _combine_body(p_ref, acc_ref, x_ref, accn_ref):
    x = p_ref[0] + p_ref[1]
    x_ref[...] = x
    accn_ref[...] = acc_ref[...] + x


_RB = 2000

_tc_combine = pl.pallas_call(
    _combine_body,
    grid=(N_NODES // _RB,),
    in_specs=[
        pl.BlockSpec((NC, _RB, EMB_DIM), lambda i: (0, i, 0)),
        pl.BlockSpec((_RB, EMB_DIM), lambda i: (i, 0)),
    ],
    out_specs=[
        pl.BlockSpec((_RB, EMB_DIM), lambda i: (i, 0)),
        pl.BlockSpec((_RB, EMB_DIM), lambda i: (i, 0)),
    ],
    out_shape=[
        jax.ShapeDtypeStruct((N_NODES, EMB_DIM), jnp.float32),
        jax.ShapeDtypeStruct((N_NODES, EMB_DIM), jnp.float32),
    ],
)


def _final_body(p_ref, acc_ref, fin_ref):
    fin_ref[...] = (acc_ref[...] + p_ref[0] + p_ref[1]) * (1.0 / (N_LAYERS + 1))


_tc_final = pl.pallas_call(
    _final_body,
    grid=(N_NODES // _RB,),
    in_specs=[
        pl.BlockSpec((NC, _RB, EMB_DIM), lambda i: (0, i, 0)),
        pl.BlockSpec((_RB, EMB_DIM), lambda i: (i, 0)),
    ],
    out_specs=pl.BlockSpec((_RB, EMB_DIM), lambda i: (i, 0)),
    out_shape=jax.ShapeDtypeStruct((N_NODES, EMB_DIM), jnp.float32),
)


def _dots_body(u_ref, p_ref, n_ref, ps_ref, ns_ref):
    u = u_ref[...]
    ps_ref[...] = jnp.sum(u * p_ref[...], axis=1, keepdims=True)
    ns_ref[...] = jnp.sum(u * n_ref[...], axis=1, keepdims=True)


_DB = 512

_tc_dots = pl.pallas_call(
    _dots_body,
    grid=(BATCH // _DB,),
    in_specs=[
        pl.BlockSpec((_DB, EMB_DIM), lambda i: (i, 0)),
        pl.BlockSpec((_DB, EMB_DIM), lambda i: (i + GATH_TOTAL // (3 * _DB), 0)),
        pl.BlockSpec((_DB, EMB_DIM), lambda i: (i + 2 * GATH_TOTAL // (3 * _DB), 0)),
    ],
    out_specs=[
        pl.BlockSpec((_DB, 1), lambda i: (i, 0)),
        pl.BlockSpec((_DB, 1), lambda i: (i, 0)),
    ],
    out_shape=[
        jax.ShapeDtypeStruct((BATCH, 1), jnp.float32),
        jax.ShapeDtypeStruct((BATCH, 1), jnp.float32),
    ],
)


def kernel(users, pos, neg, user_emb, item_emb, adj_src, adj_dst, adj_val):
    x0 = jnp.concatenate([user_emb, item_emb], axis=0)
    pad = E_PAD - N_EDGES
    src = jnp.pad(adj_src.astype(jnp.int32), (0, pad))
    dst = jnp.pad(adj_dst.astype(jnp.int32), (0, pad))
    val = jnp.pad(adj_val, (0, pad))

    x = x0
    acc = x0
    for layer in range(N_LAYERS):
        p = _sc_propagate(x, src, dst, val)
        if layer < N_LAYERS - 1:
            x, acc = _tc_combine(p, acc)
        else:
            final = _tc_final(p, acc)

    idx = jnp.concatenate([users.astype(jnp.int32),
                           pos.astype(jnp.int32) + N_USERS,
                           neg.astype(jnp.int32) + N_USERS])
    rows = _sc_gather(final, idx)
    ps, ns = _tc_dots(rows, rows, rows)
    return (ps[:, 0], ns[:, 0])
